# baseline (speedup 1.0000x reference)
import jax
import jax.numpy as jnp
from jax import lax
from jax.experimental import pallas as pl
from jax.experimental.pallas import tpu as pltpu
from jax.experimental.pallas import tpu_sc as plsc

ACTION_SIZE = 1000
BATCH = 16384
LOC_LEN = 5
DIM = 16
NC = 2
NS = 16
NW = NC * NS
BPW = BATCH // NW
GCHUNK = 128
NBLK = BPW // GCHUNK


def _sc_body(loc_hbm, wt_hbm, bias_hbm, out_hbm, idx_v, acc_v, out_v,
             bias_v, wt_sp, sems):
    sid = lax.axis_index("s")
    wid = sid * NC + lax.axis_index("c")
    base = wid * BPW

    @pl.when(sid == 0)
    def _stage_wt():
        pltpu.sync_copy(wt_hbm, wt_sp)

    cb = pltpu.async_copy(bias_hbm, bias_v, sems.at[0])
    ci = pltpu.async_copy(loc_hbm.at[:, pl.ds(base, BPW)], idx_v,
                          sems.at[1])
    cb.wait()
    ci.wait()

    @plsc.parallel_loop(BPW // DIM, LOC_LEN * (BPW // DIM), 1, unroll=4)
    def _off_body(c):
        j = c // (BPW // DIM)
        s = (c % (BPW // DIM)) * DIM
        idx_v[j, pl.ds(s, DIM)] = idx_v[j, pl.ds(s, DIM)] + j * ACTION_SIZE

    bias_vec = bias_v[...]

    @plsc.parallel_loop(0, BPW, 1, unroll=8)
    def _fill_body(i):
        acc_v[i] = bias_vec

    plsc.subcore_barrier()
    for b in range(NBLK):
        def _g_body(j, _, b=b):
            pltpu.async_copy(
                wt_sp.at[idx_v.at[j, pl.ds(b * GCHUNK, GCHUNK)]],
                acc_v.at[pl.ds(b * GCHUNK, GCHUNK), :],
                sems.at[b], add=True)
            return 0

        lax.fori_loop(0, LOC_LEN, _g_body, 0)

    row_idx = lax.iota(jnp.int32, DIM)

    for b in range(NBLK):
        def _d_body(j, _, b=b):
            pltpu.make_async_copy(wt_hbm.at[pl.ds(0, GCHUNK), :],
                                  acc_v.at[pl.ds(0, GCHUNK), :],
                                  sems.at[b]).wait()
            return 0

        lax.fori_loop(0, LOC_LEN, _d_body, 0)

        @plsc.parallel_loop(b * GCHUNK, (b + 1) * GCHUNK, 1, unroll=16,
                            carry=jnp.zeros((DIM,), jnp.int32) + b * GCHUNK)
        def _sum_body(i, colv):
            acc = jnp.maximum(acc_v[i], 0.0)
            plsc.store_scatter(out_v, [row_idx, colv], acc)
            return colv + 1

    pltpu.sync_copy(out_v, out_hbm.at[:, pl.ds(base, BPW)])


@jax.jit
def _position_encode(loc_t, wt, bias):
    mesh = plsc.VectorSubcoreMesh(core_axis_name="c", subcore_axis_name="s")
    kern = pl.kernel(
        _sc_body,
        out_type=jax.ShapeDtypeStruct((DIM, BATCH), jnp.float32),
        mesh=mesh,
        scratch_types=[
            pltpu.VMEM((LOC_LEN, BPW), jnp.int32),
            pltpu.VMEM((BPW, DIM), jnp.float32),
            pltpu.VMEM((DIM, BPW), jnp.float32),
            pltpu.VMEM((DIM,), jnp.float32),
            pltpu.VMEM_SHARED((LOC_LEN * ACTION_SIZE, DIM), jnp.float32),
            pltpu.SemaphoreType.DMA((NBLK,)),
        ],
        compiler_params=pltpu.CompilerParams(use_tc_tiling_on_sc=False,
                                             needs_layout_passes=False),
    )
    return kern(loc_t, wt, bias).T


def kernel(loc, W, bias):
    loc = loc.astype(jnp.int32)
    loc_t = loc.T
    wt = W.T.reshape(LOC_LEN * ACTION_SIZE, DIM)
    return _position_encode(loc_t, wt, bias)


# speedup vs baseline: 1.0932x; 1.0932x over previous
import jax
import jax.numpy as jnp
from jax import lax
from jax.experimental import pallas as pl
from jax.experimental.pallas import tpu as pltpu
from jax.experimental.pallas import tpu_sc as plsc

ACTION_SIZE = 1000
BATCH = 16384
LOC_LEN = 5
DIM = 16
NC = 2
NS = 16
NW = NC * NS
BPW = BATCH // NW
GCHUNK = 128
NBLK = BPW // GCHUNK


def _sc_body(loc_hbm, wt_hbm, bias_hbm, out_hbm, idx_v, acc_v, out_v,
             bias_v, wt_sp, sems):
    sid = lax.axis_index("s")
    wid = sid * NC + lax.axis_index("c")
    base = wid * BPW

    @pl.when(sid == 0)
    def _stage_wt():
        pltpu.sync_copy(wt_hbm, wt_sp)

    cb = pltpu.async_copy(bias_hbm, bias_v, sems.at[0])
    ci = pltpu.async_copy(loc_hbm.at[:, pl.ds(base, BPW)], idx_v,
                          sems.at[1])
    cb.wait()
    ci.wait()

    @plsc.parallel_loop(BPW // DIM, LOC_LEN * (BPW // DIM), 1, unroll=4)
    def _off_body(c):
        j = c // (BPW // DIM)
        s = (c % (BPW // DIM)) * DIM
        idx_v[j, pl.ds(s, DIM)] = idx_v[j, pl.ds(s, DIM)] + j * ACTION_SIZE

    bias_vec = bias_v[...]

    @plsc.parallel_loop(0, BPW, 1, unroll=8)
    def _fill_body(i):
        acc_v[i] = bias_vec

    plsc.subcore_barrier()
    for b in range(NBLK):
        def _g_body(j, _, b=b):
            pltpu.async_copy(
                wt_sp.at[idx_v.at[j, pl.ds(b * GCHUNK, GCHUNK)]],
                acc_v.at[pl.ds(b * GCHUNK, GCHUNK), :],
                sems.at[b], add=True)
            return 0

        lax.fori_loop(0, LOC_LEN, _g_body, 0)

    kvec = lax.iota(jnp.int32, DIM)
    base_vec = (kvec // 8) * (NBLK * 8 * GCHUNK) + lax.rem(kvec, 8) * GCHUNK

    for b in range(NBLK):
        def _d_body(j, _, b=b):
            pltpu.make_async_copy(wt_hbm.at[pl.ds(0, GCHUNK), :],
                                  acc_v.at[pl.ds(0, GCHUNK), :],
                                  sems.at[b]).wait()
            return 0

        lax.fori_loop(0, LOC_LEN, _d_body, 0)

        @plsc.parallel_loop(b * GCHUNK, (b + 1) * GCHUNK, 1, unroll=8,
                            carry=base_vec + b * (8 * GCHUNK) - b * GCHUNK)
        def _sum_body(i, colv):
            acc = jnp.maximum(acc_v[i], 0.0)
            plsc.store_scatter(out_v, [colv + i], acc)
            return colv

    half = NBLK * 8 * GCHUNK
    pltpu.sync_copy(out_v.at[pl.ds(0, half)],
                    out_hbm.at[pl.ds(wid * half, half)])
    pltpu.sync_copy(out_v.at[pl.ds(half, half)],
                    out_hbm.at[pl.ds(NW * half + wid * half, half)])


@jax.jit
def _position_encode(loc_t, wt, bias):
    mesh = plsc.VectorSubcoreMesh(core_axis_name="c", subcore_axis_name="s")
    kern = pl.kernel(
        _sc_body,
        out_type=jax.ShapeDtypeStruct((BATCH * DIM,), jnp.float32),
        mesh=mesh,
        scratch_types=[
            pltpu.VMEM((LOC_LEN, BPW), jnp.int32),
            pltpu.VMEM((BPW, DIM), jnp.float32),
            pltpu.VMEM((BPW * DIM,), jnp.float32),
            pltpu.VMEM((DIM,), jnp.float32),
            pltpu.VMEM_SHARED((LOC_LEN * ACTION_SIZE, DIM), jnp.float32),
            pltpu.SemaphoreType.DMA((NBLK,)),
        ],
        compiler_params=pltpu.CompilerParams(use_tc_tiling_on_sc=False,
                                             needs_layout_passes=False),
    )
    o = kern(loc_t, wt, bias)
    o = o.reshape(2, BATCH // 128, 8, 128).transpose(0, 2, 1, 3)
    return o.reshape(DIM, BATCH).T


def kernel(loc, W, bias):
    loc = loc.astype(jnp.int32)
    loc_t = loc.T
    wt = W.T.reshape(LOC_LEN * ACTION_SIZE, DIM)
    return _position_encode(loc_t, wt, bias)
